# baseline (speedup 1.0000x reference)
import functools

import jax
import jax.numpy as jnp
from jax import lax
from jax.experimental import pallas as pl
from jax.experimental.pallas import tpu as pltpu
from jax.experimental.pallas import tpu_sc as plsc

_NUM_CORES = 2
_NUM_SUBCORES = 16
_NUM_WORKERS = _NUM_CORES * _NUM_SUBCORES
_L = 16

_GDN = lax.GatherDimensionNumbers(
    offset_dims=(), collapsed_slice_dims=(0,), start_index_map=(0,))

_BATCH = 16384
_SEQ = 200
_VOCAB = 4
_EMB = 32

_BT = _BATCH // 128
_ET = _EMB // 8
_NQ = 8
_QB = _BATCH // _NQ
_QT = _QB // 128
_UNITS = _SEQ * _NQ
_UPW = _UNITS // _NUM_WORKERS


@functools.partial(
    pl.kernel,
    out_type=jax.ShapeDtypeStruct((_SEQ, _ET, _BT, 8, 128), jnp.float32),
    mesh=plsc.VectorSubcoreMesh(
        core_axis_name="c", subcore_axis_name="s",
        num_cores=_NUM_CORES, num_subcores=_NUM_SUBCORES),
    scratch_types=[
        pltpu.VMEM((_EMB, 128), jnp.float32),
        pltpu.VMEM((2, _QB), jnp.int32),
        pltpu.VMEM((_ET, _QT, 8, 128), jnp.float32),
        pltpu.SemaphoreType.DMA,
        pltpu.SemaphoreType.DMA,
        pltpu.SemaphoreType.DMA,
        pltpu.SemaphoreType.DMA,
        pltpu.SemaphoreType.DMA,
        pltpu.SemaphoreType.DMA,
    ],
    compiler_params=pltpu.CompilerParams(needs_layout_passes=False),
)
def _sc_embed(idx_hbm, wt_hbm, out_hbm, tab_v, idx_v, obuf,
              sem0, sem1, sem2, sem3, isem0, isem1):
    wid = lax.axis_index("s") * _NUM_CORES + lax.axis_index("c")
    u0 = wid * _UPW
    sems = [sem0, sem1, sem2, sem3]
    isems = [isem0, isem1]

    pltpu.sync_copy(wt_hbm, tab_v)
    tbe = [tab_v[e, pl.ds(0, _L)] for e in range(_EMB)]

    def idx_slice(uid):
        return idx_hbm.at[uid >> 3, 0, pl.ds((uid & 7) * _QB, _QB)]

    pltpu.async_copy(idx_slice(u0), idx_v.at[0], isem0)

    def unit_body(u, carry):
        uid = u0 + u
        s = uid >> 3
        q = uid & 7
        ub = u & 1

        def wait_idx(ref, sem):
            pltpu.make_async_copy(idx_slice(uid), ref, sem).wait()

        @pl.when(ub == 0)
        def _():
            wait_idx(idx_v.at[0], isem0)

        @pl.when(ub == 1)
        def _():
            wait_idx(idx_v.at[1], isem1)

        @pl.when(u + 1 < _UPW)
        def _():
            @pl.when(ub == 0)
            def _():
                pltpu.async_copy(idx_slice(uid + 1), idx_v.at[1], isem1)

            @pl.when(ub == 1)
            def _():
                pltpu.async_copy(idx_slice(uid + 1), idx_v.at[0], isem0)

        for et in range(_ET):
            dst = out_hbm.at[s, et, pl.ds(q * _QT, _QT)]

            @pl.when(u > 0)
            def _():
                pltpu.make_async_copy(obuf.at[et], dst, sems[et]).wait()

            @plsc.parallel_loop(0, _QB // _L, step=1, unroll=2)
            def group_body(gi):
                idxv = idx_v[ub, pl.ds(gi * _L, _L)]
                btl = gi >> 3
                bil0 = (gi & 7) * _L
                for ei in range(8):
                    e = et * 8 + ei
                    col = lax.gather(
                        tbe[e], idxv[:, None], _GDN, (1,),
                        mode=lax.GatherScatterMode.PROMISE_IN_BOUNDS)
                    obuf[et, btl, ei, pl.ds(bil0, _L)] = col

            pltpu.async_copy(obuf.at[et], dst, sems[et])
        return carry

    lax.fori_loop(0, _UPW, unit_body, 0)

    last = u0 + _UPW - 1
    s_l = last >> 3
    q_l = last & 7
    for et in range(_ET):
        pltpu.make_async_copy(
            obuf.at[et], out_hbm.at[s_l, et, pl.ds(q_l * _QT, _QT)],
            sems[et]).wait()


def kernel(gearShifter, W_gearShifter):
    idx_t = jnp.transpose(gearShifter.astype(jnp.int32), (1, 2, 0))
    sel = jnp.eye(_VOCAB, 128, dtype=W_gearShifter.dtype)
    wt = lax.dot_general(W_gearShifter, sel, (((0,), (0,)), ((), ())),
                         precision=lax.Precision.HIGHEST)
    out5 = _sc_embed(idx_t, wt)
    out = jnp.transpose(out5, (2, 4, 0, 1, 3)).reshape(
        _BATCH, _SEQ, 1, _EMB)
    return out


# speedup vs baseline: 1.0205x; 1.0205x over previous
import functools

import jax
import jax.numpy as jnp
from jax import lax
from jax.experimental import pallas as pl
from jax.experimental.pallas import tpu as pltpu
from jax.experimental.pallas import tpu_sc as plsc

_NUM_CORES = 2
_NUM_SUBCORES = 16
_NUM_WORKERS = _NUM_CORES * _NUM_SUBCORES
_L = 16

_GDN = lax.GatherDimensionNumbers(
    offset_dims=(), collapsed_slice_dims=(0,), start_index_map=(0,))

_BATCH = 16384
_SEQ = 200
_VOCAB = 4
_EMB = 32

_BT = _BATCH // 128
_ET = _EMB // 8
_QB = _BATCH // 4
_UNITS = _SEQ * 4
_UPW = _UNITS // _NUM_WORKERS


@functools.partial(
    pl.kernel,
    out_type=jax.ShapeDtypeStruct((_SEQ, _ET, _BT, 8, 128), jnp.float32),
    mesh=plsc.VectorSubcoreMesh(
        core_axis_name="c", subcore_axis_name="s",
        num_cores=_NUM_CORES, num_subcores=_NUM_SUBCORES),
    scratch_types=[
        pltpu.VMEM((_EMB, 128), jnp.float32),
        pltpu.VMEM((2, _QB), jnp.int32),
        pltpu.VMEM((3, 32, 8, 128), jnp.float32),
        pltpu.SemaphoreType.DMA,
        pltpu.SemaphoreType.DMA,
        pltpu.SemaphoreType.DMA,
    ],
    compiler_params=pltpu.CompilerParams(needs_layout_passes=False),
)
def _sc_embed(idx_hbm, wt_hbm, out_hbm, tab_v, idx_v, obuf,
              osem, isem0, isem1):
    wid = lax.axis_index("s") * _NUM_CORES + lax.axis_index("c")
    u0 = wid * _UPW

    pltpu.sync_copy(wt_hbm, tab_v)
    tbe = [tab_v[e, pl.ds(0, _L)] for e in range(_EMB)]

    def idx_slice(uid):
        return idx_hbm.at[uid >> 2, 0, pl.ds((uid & 3) * _QB, _QB)]

    pltpu.async_copy(idx_slice(u0), idx_v.at[0], isem0)

    def unit_body(u, carry):
        uid = u0 + u
        s = uid >> 2
        q = uid & 3
        ub = u & 1

        def wait_idx(ref, sem):
            pltpu.make_async_copy(idx_slice(uid), ref, sem).wait()

        @pl.when(ub == 0)
        def _():
            wait_idx(idx_v.at[0], isem0)

        @pl.when(ub == 1)
        def _():
            wait_idx(idx_v.at[1], isem1)

        @pl.when(u + 1 < _UPW)
        def _():
            @pl.when(ub == 0)
            def _():
                pltpu.async_copy(idx_slice(uid + 1), idx_v.at[1], isem1)

            @pl.when(ub == 1)
            def _():
                pltpu.async_copy(idx_slice(uid + 1), idx_v.at[0], isem0)

        for et in range(_ET):
            p = (u * _ET + et) - ((u * _ET + et) // 3) * 3
            dst = out_hbm.at[s, et, pl.ds(q * 32, 32)]

            def wait_prev():
                pltpu.make_async_copy(obuf.at[0], dst, osem).wait()

            if et == 3:
                wait_prev()
            else:
                @pl.when(u > 0)
                def _():
                    wait_prev()

            @plsc.parallel_loop(0, _QB // _L, step=1, unroll=2)
            def group_body(gi):
                idxv = idx_v[ub, pl.ds(gi * _L, _L)]
                btl = gi >> 3
                bil0 = (gi & 7) * _L
                for ei in range(8):
                    e = et * 8 + ei
                    col = lax.gather(
                        tbe[e], idxv[:, None], _GDN, (1,),
                        mode=lax.GatherScatterMode.PROMISE_IN_BOUNDS)
                    obuf[p, btl, ei, pl.ds(bil0, _L)] = col

            pltpu.async_copy(obuf.at[p], dst, osem)
        return carry

    lax.fori_loop(0, _UPW, unit_body, 0)

    last = u0 + _UPW - 1
    s_l = last >> 2
    q_l = last & 3
    for _ in range(3):
        pltpu.make_async_copy(
            obuf.at[0], out_hbm.at[s_l, 3, pl.ds(q_l * 32, 32)],
            osem).wait()


def kernel(gearShifter, W_gearShifter):
    idx_t = jnp.transpose(gearShifter.astype(jnp.int32), (1, 2, 0))
    sel = jnp.eye(_VOCAB, 128, dtype=W_gearShifter.dtype)
    wt = lax.dot_general(W_gearShifter, sel, (((0,), (0,)), ((), ())),
                         precision=lax.Precision.HIGHEST)
    out5 = _sc_embed(idx_t, wt)
    out = jnp.transpose(out5, (2, 4, 0, 1, 3)).reshape(
        _BATCH, _SEQ, 1, _EMB)
    return out
